# baseline (speedup 1.0000x reference)
import math

import jax
import jax.numpy as jnp
from jax.experimental import pallas as pl

_ROWS = 8


def _avg_floor(lo, hi):
    return (lo >> 1) + (hi >> 1) + (lo & hi & 1)


def _key_of_bits(u):
    return jnp.where(u < 0, u ^ jnp.int32(0x7FFFFFFF), u)


def _sampler_kernel(x_ref, p_ref, k_ref, a_ref, m_ref, o_ref):
    x = x_ref[...]
    R, V = x.shape
    p = p_ref[...]
    kk = k_ref[...]
    a = a_ref[...]
    m = m_ref[...]

    mx = jnp.max(x, axis=1, keepdims=True)
    ex = jnp.exp(x - mx)
    z = jnp.sum(ex, axis=1, keepdims=True)
    inv_z = 1.0 / z
    pmax = inv_z
    thr = jnp.maximum(pmax * m, pmax * pmax * a)
    pz = p * z

    key = _key_of_bits(jax.lax.bitcast_convert_type(x, jnp.int32))

    km1 = kk - 1
    kkf = kk.astype(jnp.float32)

    hi0 = _key_of_bits(jax.lax.bitcast_convert_type(mx, jnp.int32))

    mean = jnp.sum(x, axis=1, keepdims=True) * (1.0 / V)
    lo_est = mean + 2.0
    lo_key = _key_of_bits(jax.lax.bitcast_convert_type(lo_est, jnp.int32))
    cnt0 = jnp.sum((key > lo_key).astype(jnp.float32), axis=1, keepdims=True)
    lo0 = jnp.where(cnt0 >= kkf, lo_key,
                    jnp.full_like(hi0, jnp.iinfo(jnp.int32).min))

    km1f = km1.astype(jnp.float32)

    def cond_fn(carry):
        i, lo, hi, cl, ch, esh = carry
        return (i < 32) & jnp.any((cl - ch > 1.0) & (hi > lo + 1))

    def body(carry):
        i, lo, hi, cl, ch, esh = carry
        mid = _avg_floor(lo, hi)
        gtf = (key > mid).astype(jnp.float32)
        cnt = jnp.sum(gtf, axis=1, keepdims=True)
        es = jax.lax.cond(
            jnp.any(cnt <= km1f),
            lambda: jnp.sum(gtf * ex, axis=1, keepdims=True),
            lambda: jnp.zeros_like(cnt))
        ok = (cnt <= km1f) & (es <= pz)
        new_lo = jnp.where(ok, lo, mid)
        new_cl = jnp.where(ok, cl, cnt)
        new_hi = jnp.where(ok, mid, hi)
        new_ch = jnp.where(ok, cnt, ch)
        new_esh = jnp.where(ok, es, esh)
        return i + 1, new_lo, new_hi, new_cl, new_ch, new_esh

    cl0 = jnp.where(cnt0 >= kkf, cnt0, jnp.full_like(cnt0, float(V)))
    ch0 = jnp.zeros_like(cnt0)
    _, lo_f, hi_f, cl_f, ch_f, esh_f = jax.lax.while_loop(
        cond_fn, body, (jnp.int32(0), lo0, hi0, cl0, ch0,
                        jnp.zeros_like(cnt0)))

    in_bracket = (key > lo_f) & (key <= hi_f)
    t = jnp.max(jnp.where(in_bracket, key,
                          jnp.iinfo(jnp.int32).min), axis=1, keepdims=True)

    gt = key > t
    eq = key == t
    c_star = ch_f
    s_star = esh_f
    tie_cnt = cl_f - ch_f

    t_val = jax.lax.bitcast_convert_type(_key_of_bits(t), jnp.float32)
    e_t = jnp.exp(t_val - mx)

    q_k = kkf - c_star
    q_p = jnp.floor((pz - s_star) / jnp.maximum(e_t, 1e-38)) + 1.0
    q = jnp.minimum(q_k, jnp.clip(q_p, 0.0, float(2 ** 30)))

    idx = jax.lax.broadcasted_iota(jnp.int32, (R, V), 1)
    iters2 = max(1, int(math.ceil(math.log2(V + 1))))

    def tie_search(_):
        def body2(_, carry):
            lo, hi = carry
            mid = _avg_floor(lo, hi)
            cnt = jnp.sum((eq & (idx <= mid)).astype(jnp.float32), axis=1,
                          keepdims=True)
            ok = cnt >= q
            lo = jnp.where(ok, lo, mid)
            hi = jnp.where(ok, mid, hi)
            return lo, hi

        lo2 = jnp.full_like(t, -1)
        hi2 = jnp.full_like(t, V - 1)
        _, res = jax.lax.fori_loop(0, iters2, body2, (lo2, hi2))
        return res

    bound_i = jax.lax.cond(jnp.all(q >= tie_cnt),
                           lambda _: jnp.full_like(t, V - 1),
                           tie_search, 0)
    tie_keep = eq & (idx <= bound_i)

    cond1 = ex * inv_z >= thr

    keep = cond1 & (gt | tie_keep)
    o_ref[...] = jnp.where(keep, x, -jnp.inf)


def _build_call(nrows, vocab, rows_per_block, interpret=False):
    grid = nrows // rows_per_block
    vec_spec = pl.BlockSpec((rows_per_block, 1), lambda i: (i, 0))
    return pl.pallas_call(
        _sampler_kernel,
        grid=(grid,),
        in_specs=[
            pl.BlockSpec((rows_per_block, vocab), lambda i: (i, 0)),
            vec_spec, vec_spec, vec_spec, vec_spec,
        ],
        out_specs=pl.BlockSpec((rows_per_block, vocab), lambda i: (i, 0)),
        out_shape=jax.ShapeDtypeStruct((nrows, vocab), jnp.float32),
        interpret=interpret,
    )


def kernel(logits, p, k, a, m):
    nrows, vocab = logits.shape
    rows = _ROWS if nrows % _ROWS == 0 else 1
    call = _build_call(nrows, vocab, rows)
    return call(
        logits,
        p.reshape(nrows, 1).astype(jnp.float32),
        k.reshape(nrows, 1).astype(jnp.int32),
        a.reshape(nrows, 1).astype(jnp.float32),
        m.reshape(nrows, 1).astype(jnp.float32),
    )


# speedup vs baseline: 1.4597x; 1.4597x over previous
import math

import jax
import jax.numpy as jnp
from jax.experimental import pallas as pl

_ROWS = 8


def _avg_floor(lo, hi):
    return (lo >> 1) + (hi >> 1) + (lo & hi & 1)


def _key_of_bits(u):
    return jnp.where(u < 0, u ^ jnp.int32(0x7FFFFFFF), u)


def _sampler_kernel(x_ref, p_ref, k_ref, a_ref, m_ref, o_ref):
    x = x_ref[...]
    R, V = x.shape
    p = p_ref[...]
    kk = k_ref[...]
    a = a_ref[...]
    m = m_ref[...]

    mx = jnp.max(x, axis=1, keepdims=True)
    ex = jnp.exp(x - mx)
    z = jnp.sum(ex, axis=1, keepdims=True)
    inv_z = 1.0 / z
    pmax = inv_z
    thr = jnp.maximum(pmax * m, pmax * pmax * a)
    pz = p * z

    key = _key_of_bits(jax.lax.bitcast_convert_type(x, jnp.int32))

    km1 = kk - 1
    kkf = kk.astype(jnp.float32)

    hi0 = _key_of_bits(jax.lax.bitcast_convert_type(mx, jnp.int32))

    mean = jnp.sum(x, axis=1, keepdims=True) * (1.0 / V)
    lo_est = mean + 2.0
    lo_key = _key_of_bits(jax.lax.bitcast_convert_type(lo_est, jnp.int32))
    cnt0 = jnp.sum((key > lo_key).astype(jnp.float32), axis=1, keepdims=True)
    lo0 = jnp.where(cnt0 >= kkf, lo_key,
                    jnp.full_like(hi0, jnp.iinfo(jnp.int32).min))

    km1f = km1.astype(jnp.float32)

    def cond_fn(carry):
        i, lo, hi, cl, ch, esh = carry
        return (i < 32) & jnp.any((cl - ch > 1.0) & (hi > lo + 1))

    def body(carry):
        i, lo, hi, cl, ch, esh = carry
        mid = _avg_floor(lo, hi)
        gtf = (key > mid).astype(jnp.float32)
        cnt = jnp.sum(gtf, axis=1, keepdims=True)
        es = jnp.sum(gtf * ex, axis=1, keepdims=True)
        ok = (cnt <= km1f) & (es <= pz)
        new_lo = jnp.where(ok, lo, mid)
        new_cl = jnp.where(ok, cl, cnt)
        new_hi = jnp.where(ok, mid, hi)
        new_ch = jnp.where(ok, cnt, ch)
        new_esh = jnp.where(ok, es, esh)
        return i + 1, new_lo, new_hi, new_cl, new_ch, new_esh

    cl0 = jnp.where(cnt0 >= kkf, cnt0, jnp.full_like(cnt0, float(V)))
    ch0 = jnp.zeros_like(cnt0)
    _, lo_f, hi_f, cl_f, ch_f, esh_f = jax.lax.while_loop(
        cond_fn, body, (jnp.int32(0), lo0, hi0, cl0, ch0,
                        jnp.zeros_like(cnt0)))

    in_bracket = (key > lo_f) & (key <= hi_f)
    t = jnp.max(jnp.where(in_bracket, key,
                          jnp.iinfo(jnp.int32).min), axis=1, keepdims=True)

    gt = key > t
    eq = key == t
    c_star = ch_f
    s_star = esh_f
    tie_cnt = cl_f - ch_f

    t_val = jax.lax.bitcast_convert_type(_key_of_bits(t), jnp.float32)
    e_t = jnp.exp(t_val - mx)

    q_k = kkf - c_star
    q_p = jnp.floor((pz - s_star) / jnp.maximum(e_t, 1e-38)) + 1.0
    q = jnp.minimum(q_k, jnp.clip(q_p, 0.0, float(2 ** 30)))

    idx = jax.lax.broadcasted_iota(jnp.int32, (R, V), 1)
    iters2 = max(1, int(math.ceil(math.log2(V + 1))))

    def tie_search(_):
        def body2(_, carry):
            lo, hi = carry
            mid = _avg_floor(lo, hi)
            cnt = jnp.sum((eq & (idx <= mid)).astype(jnp.float32), axis=1,
                          keepdims=True)
            ok = cnt >= q
            lo = jnp.where(ok, lo, mid)
            hi = jnp.where(ok, mid, hi)
            return lo, hi

        lo2 = jnp.full_like(t, -1)
        hi2 = jnp.full_like(t, V - 1)
        _, res = jax.lax.fori_loop(0, iters2, body2, (lo2, hi2))
        return res

    bound_i = jax.lax.cond(jnp.all(q >= tie_cnt),
                           lambda _: jnp.full_like(t, V - 1),
                           tie_search, 0)
    tie_keep = eq & (idx <= bound_i)

    cond1 = ex * inv_z >= thr

    keep = cond1 & (gt | tie_keep)
    o_ref[...] = jnp.where(keep, x, -jnp.inf)


def _build_call(nrows, vocab, rows_per_block, interpret=False):
    grid = nrows // rows_per_block
    vec_spec = pl.BlockSpec((rows_per_block, 1), lambda i: (i, 0))
    return pl.pallas_call(
        _sampler_kernel,
        grid=(grid,),
        in_specs=[
            pl.BlockSpec((rows_per_block, vocab), lambda i: (i, 0)),
            vec_spec, vec_spec, vec_spec, vec_spec,
        ],
        out_specs=pl.BlockSpec((rows_per_block, vocab), lambda i: (i, 0)),
        out_shape=jax.ShapeDtypeStruct((nrows, vocab), jnp.float32),
        interpret=interpret,
    )


def kernel(logits, p, k, a, m):
    nrows, vocab = logits.shape
    rows = _ROWS if nrows % _ROWS == 0 else 1
    call = _build_call(nrows, vocab, rows)
    return call(
        logits,
        p.reshape(nrows, 1).astype(jnp.float32),
        k.reshape(nrows, 1).astype(jnp.int32),
        a.reshape(nrows, 1).astype(jnp.float32),
        m.reshape(nrows, 1).astype(jnp.float32),
    )


# speedup vs baseline: 1.7269x; 1.1831x over previous
import math

import jax
import jax.numpy as jnp
from jax.experimental import pallas as pl

_ROWS = 16


def _avg_floor(lo, hi):
    return (lo >> 1) + (hi >> 1) + (lo & hi & 1)


def _key_of_bits(u):
    return jnp.where(u < 0, u ^ jnp.int32(0x7FFFFFFF), u)


def _sampler_kernel(x_ref, p_ref, k_ref, a_ref, m_ref, o_ref):
    x = x_ref[...]
    R, V = x.shape
    p = p_ref[...]
    kk = k_ref[...]
    a = a_ref[...]
    m = m_ref[...]

    mx = jnp.max(x, axis=1, keepdims=True)
    ex = jnp.exp(x - mx)
    z = jnp.sum(ex, axis=1, keepdims=True)
    inv_z = 1.0 / z
    pmax = inv_z
    thr = jnp.maximum(pmax * m, pmax * pmax * a)
    pz = p * z

    key = _key_of_bits(jax.lax.bitcast_convert_type(x, jnp.int32))

    km1 = kk - 1
    kkf = kk.astype(jnp.float32)

    hi0 = _key_of_bits(jax.lax.bitcast_convert_type(mx, jnp.int32))

    mean = jnp.sum(x, axis=1, keepdims=True) * (1.0 / V)
    lo_est = mean + 2.0
    lo_key = _key_of_bits(jax.lax.bitcast_convert_type(lo_est, jnp.int32))
    cnt0 = jnp.sum((key > lo_key).astype(jnp.float32), axis=1, keepdims=True)
    lo0 = jnp.where(cnt0 >= kkf, lo_key,
                    jnp.full_like(hi0, jnp.iinfo(jnp.int32).min))

    km1f = km1.astype(jnp.float32)

    def cond_fn(carry):
        i, lo, hi, cl, ch, esh = carry
        return (i < 32) & jnp.any((cl - ch > 1.0) & (hi > lo + 1))

    def body(carry):
        i, lo, hi, cl, ch, esh = carry
        mid = _avg_floor(lo, hi)
        gtf = (key > mid).astype(jnp.float32)
        cnt = jnp.sum(gtf, axis=1, keepdims=True)
        es = jnp.sum(gtf * ex, axis=1, keepdims=True)
        ok = (cnt <= km1f) & (es <= pz)
        new_lo = jnp.where(ok, lo, mid)
        new_cl = jnp.where(ok, cl, cnt)
        new_hi = jnp.where(ok, mid, hi)
        new_ch = jnp.where(ok, cnt, ch)
        new_esh = jnp.where(ok, es, esh)
        return i + 1, new_lo, new_hi, new_cl, new_ch, new_esh

    cl0 = jnp.where(cnt0 >= kkf, cnt0, jnp.full_like(cnt0, float(V)))
    ch0 = jnp.zeros_like(cnt0)
    _, lo_f, hi_f, cl_f, ch_f, esh_f = jax.lax.while_loop(
        cond_fn, body, (jnp.int32(0), lo0, hi0, cl0, ch0,
                        jnp.zeros_like(cnt0)))

    in_bracket = (key > lo_f) & (key <= hi_f)
    t = jnp.max(jnp.where(in_bracket, key,
                          jnp.iinfo(jnp.int32).min), axis=1, keepdims=True)

    gt = key > t
    eq = key == t
    c_star = ch_f
    s_star = esh_f
    tie_cnt = cl_f - ch_f

    t_val = jax.lax.bitcast_convert_type(_key_of_bits(t), jnp.float32)
    e_t = jnp.exp(t_val - mx)

    q_k = kkf - c_star
    q_p = jnp.floor((pz - s_star) / jnp.maximum(e_t, 1e-38)) + 1.0
    q = jnp.minimum(q_k, jnp.clip(q_p, 0.0, float(2 ** 30)))

    idx = jax.lax.broadcasted_iota(jnp.int32, (R, V), 1)
    iters2 = max(1, int(math.ceil(math.log2(V + 1))))

    def tie_search(_):
        def body2(_, carry):
            lo, hi = carry
            mid = _avg_floor(lo, hi)
            cnt = jnp.sum((eq & (idx <= mid)).astype(jnp.float32), axis=1,
                          keepdims=True)
            ok = cnt >= q
            lo = jnp.where(ok, lo, mid)
            hi = jnp.where(ok, mid, hi)
            return lo, hi

        lo2 = jnp.full_like(t, -1)
        hi2 = jnp.full_like(t, V - 1)
        _, res = jax.lax.fori_loop(0, iters2, body2, (lo2, hi2))
        return res

    bound_i = jax.lax.cond(jnp.all(q >= tie_cnt),
                           lambda _: jnp.full_like(t, V - 1),
                           tie_search, 0)
    tie_keep = eq & (idx <= bound_i)

    cond1 = ex * inv_z >= thr

    keep = cond1 & (gt | tie_keep)
    o_ref[...] = jnp.where(keep, x, -jnp.inf)


def _build_call(nrows, vocab, rows_per_block, interpret=False):
    grid = nrows // rows_per_block
    vec_spec = pl.BlockSpec((rows_per_block, 1), lambda i: (i, 0))
    return pl.pallas_call(
        _sampler_kernel,
        grid=(grid,),
        in_specs=[
            pl.BlockSpec((rows_per_block, vocab), lambda i: (i, 0)),
            vec_spec, vec_spec, vec_spec, vec_spec,
        ],
        out_specs=pl.BlockSpec((rows_per_block, vocab), lambda i: (i, 0)),
        out_shape=jax.ShapeDtypeStruct((nrows, vocab), jnp.float32),
        interpret=interpret,
    )


def kernel(logits, p, k, a, m):
    nrows, vocab = logits.shape
    rows = _ROWS if nrows % _ROWS == 0 else 1
    call = _build_call(nrows, vocab, rows)
    return call(
        logits,
        p.reshape(nrows, 1).astype(jnp.float32),
        k.reshape(nrows, 1).astype(jnp.int32),
        a.reshape(nrows, 1).astype(jnp.float32),
        m.reshape(nrows, 1).astype(jnp.float32),
    )
